# Initial kernel scaffold; baseline (speedup 1.0000x reference)
import jax
import jax.numpy as jnp
from jax.experimental import pallas as pl


def kernel(xyz, feats, idx, g1, b1, g2, b2, Wq, Wkv, Wp1, Wp2, Wa1, Wa2, Wproj, Wf1, Wf2):
    raise NotImplementedError("write your pallas kernel here")


# speedup vs baseline: 9.5235x; 9.5235x over previous
import functools

import jax
import jax.numpy as jnp
from jax import lax
from jax.experimental import pallas as pl
from jax.experimental.pallas import tpu as pltpu
from jax.experimental.pallas import tpu_sc as plsc

B, N, K, DIM, HID = 2, 4096, 16, 256, 512
DIM2 = 2 * DIM
XP = 128
DIM3 = DIM2 + XP
BN = 256
NB = N // BN
BM = 1024
CH = 128
NSC, NSUB = 2, 16
NW = NSC * NSUB
CPW = N // CH


def _fold_body(wq, wa1, wkv, wp2, wqa_ref, wg_ref, wp2cat_ref):
    a1 = wa1[...]
    wqa_ref[...] = jnp.dot(wq[...], a1, preferred_element_type=jnp.float32)
    wg_ref[:, :DIM] = jnp.dot(wkv[:, :DIM], a1, preferred_element_type=jnp.float32)
    wg_ref[:, DIM:] = wkv[:, DIM:]
    wp2cat_ref[:, :DIM] = wp2[...]
    wp2cat_ref[:, DIM:] = jnp.dot(wp2[...], a1, preferred_element_type=jnp.float32)


def _pre_body(feats, xyzp, g1b1, wqa, wg, qa_ref, g_ref):
    x = feats[...]
    m = jnp.mean(x, axis=-1, keepdims=True)
    xc = x - m
    v = jnp.mean(xc * xc, axis=-1, keepdims=True)
    xn = xc * lax.rsqrt(v + 1e-5) * g1b1[0:1, :] + g1b1[1:2, :]
    qa_ref[...] = jnp.dot(xn, wqa[...], preferred_element_type=jnp.float32)
    g_ref[:, :DIM2] = jnp.dot(xn, wg[...], preferred_element_type=jnp.float32)
    g_ref[:, DIM2:] = xyzp[...]


def _gather_body(gtab, idxf, gg_out, idx_v, gbuf, sem_g):
    wid = lax.axis_index("s") * NSC + lax.axis_index("c")
    pltpu.sync_copy(idxf.at[wid], idx_v)

    def chunk(c, carry):
        base = c * CH
        isl = idx_v.at[pl.ds(base, CH)]
        pltpu.async_copy(gtab.at[isl], gbuf, sem_g).wait()
        pltpu.sync_copy(gbuf, gg_out.at[wid, pl.ds(base, CH)])
        return carry

    lax.fori_loop(0, CPW, chunk, 0)


def _attn_body(feats, xyzp, qa, gg, wp1p, wp2cat, wa2, wproj, wf1, wf2,
               g2b2, out_ref):
    f = feats[...]
    xp = xyzp[...]
    q = qa[...]
    m = jnp.full((BN, DIM), -1e30, jnp.float32)
    s = jnp.zeros((BN, DIM), jnp.float32)
    acc = jnp.zeros((BN, DIM), jnp.float32)
    for k in range(K):
        slab = gg[0, k]
        rel = slab[:, DIM2:] - xp
        h = jnp.maximum(jnp.dot(rel, wp1p[...], preferred_element_type=jnp.float32), 0.0)
        pp = jnp.dot(h, wp2cat[...], preferred_element_type=jnp.float32)
        t = jnp.dot(jnp.maximum(q - slab[:, :DIM] + pp[:, DIM:], 0.0), wa2[...],
                    preferred_element_type=jnp.float32)
        mn = jnp.maximum(m, t)
        sc = jnp.exp(m - mn)
        e = jnp.exp(t - mn)
        s = s * sc + e
        acc = acc * sc + e * (slab[:, DIM:DIM2] + pp[:, :DIM])
        m = mn
    out = acc / s
    y = f + jnp.dot(out, wproj[...], preferred_element_type=jnp.float32)
    mu = jnp.mean(y, axis=-1, keepdims=True)
    yc = y - mu
    var = jnp.mean(yc * yc, axis=-1, keepdims=True)
    ln = yc * lax.rsqrt(var + 1e-5) * g2b2[0:1, :] + g2b2[1:2, :]
    z = y + jnp.dot(jnp.maximum(jnp.dot(ln, wf1[...], preferred_element_type=jnp.float32), 0.0),
                    wf2[...], preferred_element_type=jnp.float32)
    out_ref[...] = z


def _fold_call(wq, wa1, wkv, wp2):
    return pl.pallas_call(
        _fold_body,
        out_shape=(
            jax.ShapeDtypeStruct((DIM, DIM), jnp.float32),
            jax.ShapeDtypeStruct((DIM, DIM2), jnp.float32),
            jax.ShapeDtypeStruct((DIM, DIM2), jnp.float32),
        ),
    )(wq, wa1, wkv, wp2)


def _pre_call(feats2, xyzp, g1b1, wqa, wg):
    nblk = (B * N) // BM
    return pl.pallas_call(
        _pre_body,
        grid=(nblk,),
        in_specs=[
            pl.BlockSpec((BM, DIM), lambda i: (i, 0)),
            pl.BlockSpec((BM, XP), lambda i: (i, 0)),
            pl.BlockSpec((2, DIM), lambda i: (0, 0)),
            pl.BlockSpec((DIM, DIM), lambda i: (0, 0)),
            pl.BlockSpec((DIM, DIM2), lambda i: (0, 0)),
        ],
        out_specs=(
            pl.BlockSpec((BM, DIM), lambda i: (i, 0)),
            pl.BlockSpec((BM, DIM3), lambda i: (i, 0)),
        ),
        out_shape=(
            jax.ShapeDtypeStruct((B * N, DIM), jnp.float32),
            jax.ShapeDtypeStruct((B * N, DIM3), jnp.float32),
        ),
    )(feats2, xyzp, g1b1, wqa, wg)


def _gather_call(gtab, idxf):
    k = functools.partial(
        pl.kernel,
        mesh=plsc.VectorSubcoreMesh(core_axis_name="c", subcore_axis_name="s"),
        out_type=jax.ShapeDtypeStruct((B * K, N, DIM3), jnp.float32),
        scratch_types=[
            pltpu.VMEM((N,), jnp.int32),
            pltpu.VMEM((CH, DIM3), jnp.float32),
            pltpu.SemaphoreType.DMA,
        ],
    )(_gather_body)
    return k(gtab, idxf)


def _attn_call(feats2, xyzp, qa, gg, wp1p, wp2cat, wa2, wproj, wf1, wf2, g2b2):
    return pl.pallas_call(
        _attn_body,
        grid=(B * NB,),
        in_specs=[
            pl.BlockSpec((BN, DIM), lambda i: (i, 0)),
            pl.BlockSpec((BN, XP), lambda i: (i, 0)),
            pl.BlockSpec((BN, DIM), lambda i: (i, 0)),
            pl.BlockSpec((1, K, BN, DIM3), lambda i: (i // NB, 0, i % NB, 0)),
            pl.BlockSpec((XP, DIM), lambda i: (0, 0)),
            pl.BlockSpec((DIM, DIM2), lambda i: (0, 0)),
            pl.BlockSpec((DIM, DIM), lambda i: (0, 0)),
            pl.BlockSpec((DIM, DIM), lambda i: (0, 0)),
            pl.BlockSpec((DIM, HID), lambda i: (0, 0)),
            pl.BlockSpec((HID, DIM), lambda i: (0, 0)),
            pl.BlockSpec((2, DIM), lambda i: (0, 0)),
        ],
        out_specs=pl.BlockSpec((BN, DIM), lambda i: (i, 0)),
        out_shape=jax.ShapeDtypeStruct((B * N, DIM), jnp.float32),
    )(feats2, xyzp, qa, gg, wp1p, wp2cat, wa2, wproj, wf1, wf2, g2b2)


def kernel(xyz, feats, idx, g1, b1, g2, b2, Wq, Wkv, Wp1, Wp2, Wa1, Wa2, Wproj, Wf1, Wf2):
    feats2 = feats.reshape(B * N, DIM)
    xyzp = jnp.pad(xyz, ((0, 0), (0, 0), (0, XP - 3))).reshape(B * N, XP)
    wp1p = jnp.pad(Wp1, ((0, XP - 3), (0, 0)))
    g1b1 = jnp.stack([g1, b1])
    g2b2 = jnp.stack([g2, b2])
    idxf = (idx + (jnp.arange(B, dtype=jnp.int32) * N)[:, None, None]
            ).transpose(0, 2, 1).reshape(B * K, N)

    wqa, wg, wp2cat = _fold_call(Wq, Wa1, Wkv, Wp2)
    qa, gtab = _pre_call(feats2, xyzp, g1b1, wqa, wg)
    gg = _gather_call(gtab, idxf)
    z = _attn_call(feats2, xyzp, qa, gg.reshape(B, K, N, DIM3),
                   wp1p, wp2cat, Wa2, Wproj, Wf1, Wf2, g2b2)
    return z.reshape(B, N, DIM)
